# baseline (speedup 1.0000x reference)
import functools

import jax
import jax.numpy as jnp
from jax import lax
from jax.experimental import pallas as pl
from jax.experimental.pallas import tpu as pltpu
from jax.experimental.pallas import tpu_sc as plsc

N = 10000
NP = 10240
E = 320000
HID = 128

NC = 2
NS = 16
NW = NC * NS
EPT = E // NW
CH = 80
NCHUNK = EPT // CH
ROWS_PER_TILE = NP // NS

_MESH = plsc.VectorSubcoreMesh(
    core_axis_name="c", subcore_axis_name="s", num_cores=NC, num_subcores=NS)


def _make_sc_segsum(D, NSLOT):
  @functools.partial(
      pl.kernel,
      out_type=jax.ShapeDtypeStruct((NC, NP, D), jnp.float32),
      mesh=_MESH,
      scratch_types=[
          pltpu.VMEM((NCHUNK, CH), jnp.int32),
          pltpu.VMEM((NCHUNK, CH), jnp.int32),
          [pltpu.VMEM((CH, D), jnp.float32)] * NSLOT,
          pltpu.VMEM_SHARED((NP, D), jnp.float32),
          [pltpu.SemaphoreType.DMA] * NSLOT,
      ],
      compiler_params=pltpu.CompilerParams(use_tc_tiling_on_sc=False),
  )
  def segsum(table_hbm, src2d_hbm, dst2d_hbm, zeros_hbm, out_hbm,
             src_v, dst_v, rows, acc, sems):
    cid = lax.axis_index("c")
    sid = lax.axis_index("s")
    wid = sid * NC + cid
    r0 = sid * ROWS_PER_TILE
    c0 = wid * NCHUNK

    pltpu.sync_copy(src2d_hbm.at[pl.ds(c0, NCHUNK)], src_v)
    pltpu.sync_copy(dst2d_hbm.at[pl.ds(c0, NCHUNK)], dst_v)
    pltpu.sync_copy(zeros_hbm.at[pl.ds(r0, ROWS_PER_TILE)],
                    acc.at[pl.ds(r0, ROWS_PER_TILE)])

    def gather(c, s):
      return pltpu.async_copy(table_hbm.at[src_v.at[c]], rows[s], sems[s])

    for s in range(NSLOT):
      gather(s, s)
    plsc.subcore_barrier()

    def body(i, carry):
      for s in range(NSLOT):
        c = i * NSLOT + s
        pltpu.make_async_copy(table_hbm.at[src_v.at[c]], rows[s],
                              sems[s]).wait()
        pltpu.sync_copy(rows[s], acc.at[dst_v.at[c]], add=True)

        @pl.when(c + NSLOT < NCHUNK)
        def _():
          gather(c + NSLOT, s)

      return carry

    lax.fori_loop(0, NCHUNK // NSLOT, body, 0)
    for c in range(NCHUNK - NCHUNK % NSLOT, NCHUNK):
      s = c % NSLOT
      pltpu.make_async_copy(table_hbm.at[src_v.at[c]], rows[s], sems[s]).wait()
      pltpu.sync_copy(rows[s], acc.at[dst_v.at[c]], add=True)

    plsc.subcore_barrier()
    pltpu.sync_copy(acc.at[pl.ds(r0, ROWS_PER_TILE)],
                    out_hbm.at[cid, pl.ds(r0, ROWS_PER_TILE)])

  return segsum


_sc_segsum16 = _make_sc_segsum(16, 4)
_sc_segsum128 = _make_sc_segsum(HID, 2)


NSLOT_E = 5


@functools.partial(
    pl.kernel,
    out_type=jax.ShapeDtypeStruct((E,), jnp.float32),
    mesh=_MESH,
    scratch_types=[
        pltpu.VMEM((NCHUNK, CH), jnp.int32),
        pltpu.VMEM((NCHUNK, CH), jnp.int32),
        pltpu.VMEM((HID + 16,), jnp.float32),
        [pltpu.VMEM((CH, HID), jnp.float32)] * NSLOT_E,
        [pltpu.VMEM((CH, HID), jnp.float32)] * NSLOT_E,
        [pltpu.VMEM((CH,), jnp.float32)] * NSLOT_E,
        [pltpu.SemaphoreType.DMA] * NSLOT_E,
        [pltpu.SemaphoreType.DMA] * NSLOT_E,
    ],
    compiler_params=pltpu.CompilerParams(
        use_tc_tiling_on_sc=False, needs_layout_passes=False),
)
def _sc_edge_mlp(a_hbm, b_hbm, src2d_hbm, dst2d_hbm, w_hbm, out_hbm,
                 src_v, dst_v, w_v, a_bufs, b_bufs, o_bufs, sems_a, sems_b):
  cid = lax.axis_index("c")
  sid = lax.axis_index("s")
  wid = sid * NC + cid
  base = wid * EPT
  c0 = wid * NCHUNK

  pltpu.sync_copy(src2d_hbm.at[pl.ds(c0, NCHUNK)], src_v)
  pltpu.sync_copy(dst2d_hbm.at[pl.ds(c0, NCHUNK)], dst_v)
  pltpu.sync_copy(w_hbm, w_v)

  def gathers(c, s):
    pltpu.async_copy(a_hbm.at[src_v.at[c]], a_bufs[s], sems_a[s])
    pltpu.async_copy(b_hbm.at[dst_v.at[c]], b_bufs[s], sems_b[s])

  for s in range(NSLOT_E):
    gathers(s, s)

  w_regs = [w_v[pl.ds(j * 16, 16)] for j in range(HID // 16)]
  be2_vec = w_v[pl.ds(HID, 16)]

  def process(c, s):
    pltpu.make_async_copy(a_hbm.at[src_v.at[c]], a_bufs[s], sems_a[s]).wait()
    pltpu.make_async_copy(b_hbm.at[dst_v.at[c]], b_bufs[s], sems_b[s]).wait()

    lane = lax.iota(jnp.int32, 16)
    JU = 16

    def group_body(g, carry2):
      rows16 = lane + g * 16

      def j_body(jb, acc_):
        wv = w_v[pl.ds(jb * JU, 16)]
        for dj in range(JU):
          j = jb * JU + dj
          colj = jnp.full((16,), j, jnp.int32)
          av = plsc.load_gather(a_bufs[s], [rows16, colj])
          bv = plsc.load_gather(b_bufs[s], [rows16, colj])
          acc_ = acc_ + jnp.maximum(av + bv, 0.0) * wv[dj]
        return acc_

      gvec = lax.fori_loop(0, HID // JU, j_body, be2_vec)
      o_bufs[s][pl.ds(g * 16, 16)] = 1.0 / (1.0 + jnp.exp(-gvec))
      return carry2

    lax.fori_loop(0, CH // 16, group_body, 0)
    pltpu.sync_copy(o_bufs[s], out_hbm.at[pl.ds(base + c * CH, CH)])

  def body(i, carry):
    for s in range(NSLOT_E):
      c = i * NSLOT_E + s
      process(c, s)

      @pl.when(c + NSLOT_E < NCHUNK)
      def _():
        gathers(c + NSLOT_E, s)

    return carry

  lax.fori_loop(0, NCHUNK // NSLOT_E, body, 0)
  for c in range(NCHUNK - NCHUNK % NSLOT_E, NCHUNK):
    process(c, c % NSLOT_E)


def _tc_sage1(p0, p1, x_aug, wl1t, wr1t, b1row):
  BN = 640
  grid = (NP // BN,)

  def body(p0_ref, p1_ref, x_ref, wl_ref, wr_ref, b_ref, o_ref):
    p = p0_ref[...] + p1_ref[...]
    deg = jnp.maximum(p[:, 7:8], 1.0)
    agg = p / deg
    acc = jnp.dot(agg, wl_ref[...], preferred_element_type=jnp.float32)
    acc += jnp.dot(x_ref[...], wr_ref[...], preferred_element_type=jnp.float32)
    o_ref[...] = jnp.maximum(acc + b_ref[...], 0.0)

  return pl.pallas_call(
      body,
      grid=grid,
      in_specs=[
          pl.BlockSpec((BN, 16), lambda i: (i, 0)),
          pl.BlockSpec((BN, 16), lambda i: (i, 0)),
          pl.BlockSpec((BN, 16), lambda i: (i, 0)),
          pl.BlockSpec((16, HID), lambda i: (0, 0)),
          pl.BlockSpec((16, HID), lambda i: (0, 0)),
          pl.BlockSpec((1, HID), lambda i: (0, 0)),
      ],
      out_specs=pl.BlockSpec((BN, HID), lambda i: (i, 0)),
      out_shape=jax.ShapeDtypeStruct((NP, HID), jnp.float32),
  )(p0, p1, x_aug, wl1t, wr1t, b1row)


def _tc_sage2(q0, q1, h1, p0, p1, wl2t, wr2t, b2row, wv1t, bv1row, wv2t8,
              bv2row8, we1at, we1bt, be1row):
  BN = 640
  grid = (NP // BN,)

  def body(q0_ref, q1_ref, h1_ref, p0_ref, p1_ref, wl_ref, wr_ref, b2_ref,
           wv1_ref, bv1_ref, wv2_ref, bv2_ref, wea_ref, web_ref, be1_ref,
           pos_ref, a_ref, bm_ref):
    p = p0_ref[...] + p1_ref[...]
    deg = jnp.maximum(p[:, 7:8], 1.0)
    q = (q0_ref[...] + q1_ref[...]) / deg
    h1 = h1_ref[...]
    h = jnp.dot(q, wl_ref[...], preferred_element_type=jnp.float32)
    h += jnp.dot(h1, wr_ref[...], preferred_element_type=jnp.float32)
    h += b2_ref[...]
    t = jnp.maximum(
        jnp.dot(h, wv1_ref[...], preferred_element_type=jnp.float32)
        + bv1_ref[...], 0.0)
    pos_ref[...] = jnp.dot(
        t, wv2_ref[...], preferred_element_type=jnp.float32) + bv2_ref[...]
    a_ref[...] = jnp.dot(
        h, wea_ref[...], preferred_element_type=jnp.float32) + be1_ref[...]
    bm_ref[...] = jnp.dot(
        h, web_ref[...], preferred_element_type=jnp.float32)

  return pl.pallas_call(
      body,
      grid=grid,
      in_specs=[
          pl.BlockSpec((BN, HID), lambda i: (i, 0)),
          pl.BlockSpec((BN, HID), lambda i: (i, 0)),
          pl.BlockSpec((BN, HID), lambda i: (i, 0)),
          pl.BlockSpec((BN, 16), lambda i: (i, 0)),
          pl.BlockSpec((BN, 16), lambda i: (i, 0)),
          pl.BlockSpec((HID, HID), lambda i: (0, 0)),
          pl.BlockSpec((HID, HID), lambda i: (0, 0)),
          pl.BlockSpec((1, HID), lambda i: (0, 0)),
          pl.BlockSpec((HID, HID), lambda i: (0, 0)),
          pl.BlockSpec((1, HID), lambda i: (0, 0)),
          pl.BlockSpec((HID, 8), lambda i: (0, 0)),
          pl.BlockSpec((1, 8), lambda i: (0, 0)),
          pl.BlockSpec((HID, HID), lambda i: (0, 0)),
          pl.BlockSpec((HID, HID), lambda i: (0, 0)),
          pl.BlockSpec((1, HID), lambda i: (0, 0)),
      ],
      out_specs=[
          pl.BlockSpec((BN, 8), lambda i: (i, 0)),
          pl.BlockSpec((BN, HID), lambda i: (i, 0)),
          pl.BlockSpec((BN, HID), lambda i: (i, 0)),
      ],
      out_shape=[
          jax.ShapeDtypeStruct((NP, 8), jnp.float32),
          jax.ShapeDtypeStruct((NP, HID), jnp.float32),
          jax.ShapeDtypeStruct((NP, HID), jnp.float32),
      ],
  )(q0, q1, h1, p0, p1, wl2t, wr2t, b2row, wv1t, bv1row, wv2t8, bv2row8,
    we1at, we1bt, be1row)


def kernel(x, edge_index, edge_attr, Wl1, Wr1, b1, Wl2, Wr2, b2, Wg, att_src,
           att_dst, bg, Wv1, bv1, Wv2, bv2, We1, be1, We2, be2):
  del edge_attr, Wg, att_src, att_dst, bg
  src2d = edge_index[0].reshape(E // CH, CH)
  dst2d = edge_index[1].reshape(E // CH, CH)

  ones = jnp.ones((N, 1), jnp.float32)
  zeros8 = jnp.zeros((N, 8), jnp.float32)
  x_aug = jnp.concatenate([x, ones, zeros8], axis=1)
  x_aug = jnp.pad(x_aug, ((0, NP - N), (0, 0)))

  z16 = jnp.zeros((NP, 16), jnp.float32)
  z128 = jnp.zeros((NP, HID), jnp.float32)

  pad9 = jnp.zeros((9, HID), jnp.float32)
  wl1t = jnp.concatenate([Wl1.T, pad9], axis=0)
  wr1t = jnp.concatenate([Wr1.T, pad9], axis=0)
  b1row = b1.reshape(1, HID)
  wl2t = Wl2.T
  wr2t = Wr2.T
  b2row = b2.reshape(1, HID)
  wv1t = Wv1.T
  bv1row = bv1.reshape(1, HID)
  wv2t8 = jnp.concatenate([Wv2.T, jnp.zeros((HID, 5), jnp.float32)], axis=1)
  bv2row8 = jnp.concatenate([bv2, jnp.zeros((5,), jnp.float32)]).reshape(1, 8)
  we1at = We1[:, :HID].T
  we1bt = We1[:, HID:].T
  be1row = be1.reshape(1, HID)
  w_ext = jnp.concatenate([We2.reshape(HID), jnp.broadcast_to(be2, (16,))])

  p = _sc_segsum16(x_aug, src2d, dst2d, z16)
  p0, p1 = p[0], p[1]
  h1 = _tc_sage1(p0, p1, x_aug, wl1t, wr1t, b1row)
  q = _sc_segsum128(h1, src2d, dst2d, z128)
  pos8, a_tab, b_tab = _tc_sage2(q[0], q[1], h1, p0, p1, wl2t, wr2t, b2row,
                                 wv1t, bv1row, wv2t8, bv2row8, we1at, we1bt,
                                 be1row)
  conn = _sc_edge_mlp(a_tab, b_tab, src2d, dst2d, w_ext).reshape(E, 1)

  return pos8[:N, :3], conn


# speedup vs baseline: 3.6568x; 3.6568x over previous
import functools

import jax
import jax.numpy as jnp
from jax import lax
from jax.experimental import pallas as pl
from jax.experimental.pallas import tpu as pltpu
from jax.experimental.pallas import tpu_sc as plsc

N = 10000
NP = 10240
E = 320000
HID = 128

NC = 2
NS = 16
NW = NC * NS
EPT = E // NW
CH = 80
NCHUNK = EPT // CH
ROWS_PER_TILE = NP // NS

_MESH = plsc.VectorSubcoreMesh(
    core_axis_name="c", subcore_axis_name="s", num_cores=NC, num_subcores=NS)


def _make_sc_segsum(D, NSLOT):
  @functools.partial(
      pl.kernel,
      out_type=jax.ShapeDtypeStruct((NC, NP, D), jnp.float32),
      mesh=_MESH,
      scratch_types=[
          pltpu.VMEM((NCHUNK, CH), jnp.int32),
          pltpu.VMEM((NCHUNK, CH), jnp.int32),
          [pltpu.VMEM((CH, D), jnp.float32)] * NSLOT,
          pltpu.VMEM_SHARED((NP, D), jnp.float32),
          [pltpu.SemaphoreType.DMA] * NSLOT,
      ],
      compiler_params=pltpu.CompilerParams(use_tc_tiling_on_sc=False),
  )
  def segsum(table_hbm, src2d_hbm, dst2d_hbm, zeros_hbm, out_hbm,
             src_v, dst_v, rows, acc, sems):
    cid = lax.axis_index("c")
    sid = lax.axis_index("s")
    wid = sid * NC + cid
    r0 = sid * ROWS_PER_TILE
    c0 = wid * NCHUNK

    pltpu.sync_copy(src2d_hbm.at[pl.ds(c0, NCHUNK)], src_v)
    pltpu.sync_copy(dst2d_hbm.at[pl.ds(c0, NCHUNK)], dst_v)
    pltpu.sync_copy(zeros_hbm.at[pl.ds(r0, ROWS_PER_TILE)],
                    acc.at[pl.ds(r0, ROWS_PER_TILE)])

    def gather(c, s):
      return pltpu.async_copy(table_hbm.at[src_v.at[c]], rows[s], sems[s])

    for s in range(NSLOT):
      gather(s, s)
    plsc.subcore_barrier()

    def body(i, carry):
      for s in range(NSLOT):
        c = i * NSLOT + s
        pltpu.make_async_copy(table_hbm.at[src_v.at[c]], rows[s],
                              sems[s]).wait()
        pltpu.sync_copy(rows[s], acc.at[dst_v.at[c]], add=True)

        @pl.when(c + NSLOT < NCHUNK)
        def _():
          gather(c + NSLOT, s)

      return carry

    lax.fori_loop(0, NCHUNK // NSLOT, body, 0)
    for c in range(NCHUNK - NCHUNK % NSLOT, NCHUNK):
      s = c % NSLOT
      pltpu.make_async_copy(table_hbm.at[src_v.at[c]], rows[s], sems[s]).wait()
      pltpu.sync_copy(rows[s], acc.at[dst_v.at[c]], add=True)

    plsc.subcore_barrier()
    pltpu.sync_copy(acc.at[pl.ds(r0, ROWS_PER_TILE)],
                    out_hbm.at[cid, pl.ds(r0, ROWS_PER_TILE)])

  return segsum


_sc_segsum16 = _make_sc_segsum(16, 4)
_sc_segsum128 = _make_sc_segsum(HID, 2)


NSLOT_E = 5


@functools.partial(
    pl.kernel,
    out_type=jax.ShapeDtypeStruct((E,), jnp.float32),
    mesh=_MESH,
    scratch_types=[
        pltpu.VMEM((NCHUNK, CH), jnp.int32),
        pltpu.VMEM((NCHUNK, CH), jnp.int32),
        pltpu.VMEM((HID + 16,), jnp.float32),
        [pltpu.VMEM((CH, HID), jnp.float32)] * NSLOT_E,
        [pltpu.VMEM((CH, HID), jnp.float32)] * NSLOT_E,
        [pltpu.VMEM((CH,), jnp.float32)] * NSLOT_E,
        [pltpu.SemaphoreType.DMA] * NSLOT_E,
        [pltpu.SemaphoreType.DMA] * NSLOT_E,
    ],
    compiler_params=pltpu.CompilerParams(
        use_tc_tiling_on_sc=False, needs_layout_passes=False),
)
def _sc_edge_mlp(a_hbm, b_hbm, src2d_hbm, dst2d_hbm, w_hbm, out_hbm,
                 src_v, dst_v, w_v, a_bufs, b_bufs, o_bufs, sems_a, sems_b):
  cid = lax.axis_index("c")
  sid = lax.axis_index("s")
  wid = sid * NC + cid
  base = wid * EPT
  c0 = wid * NCHUNK

  pltpu.sync_copy(src2d_hbm.at[pl.ds(c0, NCHUNK)], src_v)
  pltpu.sync_copy(dst2d_hbm.at[pl.ds(c0, NCHUNK)], dst_v)
  pltpu.sync_copy(w_hbm, w_v)

  def gathers(c, s):
    pltpu.async_copy(a_hbm.at[src_v.at[c]], a_bufs[s], sems_a[s])
    pltpu.async_copy(b_hbm.at[dst_v.at[c]], b_bufs[s], sems_b[s])

  for s in range(NSLOT_E):
    gathers(s, s)

  w_regs = [w_v[pl.ds(j * 16, 16)] for j in range(HID // 16)]
  be2_vec = w_v[pl.ds(HID, 16)]

  def process(c, s):
    pltpu.make_async_copy(a_hbm.at[src_v.at[c]], a_bufs[s], sems_a[s]).wait()
    pltpu.make_async_copy(b_hbm.at[dst_v.at[c]], b_bufs[s], sems_b[s]).wait()

    lane = lax.iota(jnp.int32, 16)

    def group_body(g, carry2):
      gvec = be2_vec
      for u in range(16):
        r = g * 16 + u
        acc = None
        for j in range(HID // 16):
          sl = pl.ds(j * 16, 16)
          t = jnp.maximum(a_bufs[s][r, sl] + b_bufs[s][r, sl], 0.0) * w_regs[j]
          acc = t if acc is None else acc + t
        gvec = jnp.where(lane == u, gvec + jnp.sum(acc), gvec)
      o_bufs[s][pl.ds(g * 16, 16)] = 1.0 / (1.0 + jnp.exp(-gvec))
      return carry2

    lax.fori_loop(0, CH // 16, group_body, 0)
    pltpu.sync_copy(o_bufs[s], out_hbm.at[pl.ds(base + c * CH, CH)])

  def body(i, carry):
    for s in range(NSLOT_E):
      c = i * NSLOT_E + s
      process(c, s)

      @pl.when(c + NSLOT_E < NCHUNK)
      def _():
        gathers(c + NSLOT_E, s)

    return carry

  lax.fori_loop(0, NCHUNK // NSLOT_E, body, 0)
  for c in range(NCHUNK - NCHUNK % NSLOT_E, NCHUNK):
    process(c, c % NSLOT_E)


def _tc_sage1(p0, p1, x_aug, wl1t, wr1t, b1row):
  BN = 640
  grid = (NP // BN,)

  def body(p0_ref, p1_ref, x_ref, wl_ref, wr_ref, b_ref, o_ref):
    p = p0_ref[...] + p1_ref[...]
    deg = jnp.maximum(p[:, 7:8], 1.0)
    agg = p / deg
    acc = jnp.dot(agg, wl_ref[...], preferred_element_type=jnp.float32)
    acc += jnp.dot(x_ref[...], wr_ref[...], preferred_element_type=jnp.float32)
    o_ref[...] = jnp.maximum(acc + b_ref[...], 0.0)

  return pl.pallas_call(
      body,
      grid=grid,
      in_specs=[
          pl.BlockSpec((BN, 16), lambda i: (i, 0)),
          pl.BlockSpec((BN, 16), lambda i: (i, 0)),
          pl.BlockSpec((BN, 16), lambda i: (i, 0)),
          pl.BlockSpec((16, HID), lambda i: (0, 0)),
          pl.BlockSpec((16, HID), lambda i: (0, 0)),
          pl.BlockSpec((1, HID), lambda i: (0, 0)),
      ],
      out_specs=pl.BlockSpec((BN, HID), lambda i: (i, 0)),
      out_shape=jax.ShapeDtypeStruct((NP, HID), jnp.float32),
  )(p0, p1, x_aug, wl1t, wr1t, b1row)


def _tc_sage2(q0, q1, h1, p0, p1, wl2t, wr2t, b2row, wv1t, bv1row, wv2t8,
              bv2row8, we1at, we1bt, be1row):
  BN = 640
  grid = (NP // BN,)

  def body(q0_ref, q1_ref, h1_ref, p0_ref, p1_ref, wl_ref, wr_ref, b2_ref,
           wv1_ref, bv1_ref, wv2_ref, bv2_ref, wea_ref, web_ref, be1_ref,
           pos_ref, a_ref, bm_ref):
    p = p0_ref[...] + p1_ref[...]
    deg = jnp.maximum(p[:, 7:8], 1.0)
    q = (q0_ref[...] + q1_ref[...]) / deg
    h1 = h1_ref[...]
    h = jnp.dot(q, wl_ref[...], preferred_element_type=jnp.float32)
    h += jnp.dot(h1, wr_ref[...], preferred_element_type=jnp.float32)
    h += b2_ref[...]
    t = jnp.maximum(
        jnp.dot(h, wv1_ref[...], preferred_element_type=jnp.float32)
        + bv1_ref[...], 0.0)
    pos_ref[...] = jnp.dot(
        t, wv2_ref[...], preferred_element_type=jnp.float32) + bv2_ref[...]
    a_ref[...] = jnp.dot(
        h, wea_ref[...], preferred_element_type=jnp.float32) + be1_ref[...]
    bm_ref[...] = jnp.dot(
        h, web_ref[...], preferred_element_type=jnp.float32)

  return pl.pallas_call(
      body,
      grid=grid,
      in_specs=[
          pl.BlockSpec((BN, HID), lambda i: (i, 0)),
          pl.BlockSpec((BN, HID), lambda i: (i, 0)),
          pl.BlockSpec((BN, HID), lambda i: (i, 0)),
          pl.BlockSpec((BN, 16), lambda i: (i, 0)),
          pl.BlockSpec((BN, 16), lambda i: (i, 0)),
          pl.BlockSpec((HID, HID), lambda i: (0, 0)),
          pl.BlockSpec((HID, HID), lambda i: (0, 0)),
          pl.BlockSpec((1, HID), lambda i: (0, 0)),
          pl.BlockSpec((HID, HID), lambda i: (0, 0)),
          pl.BlockSpec((1, HID), lambda i: (0, 0)),
          pl.BlockSpec((HID, 8), lambda i: (0, 0)),
          pl.BlockSpec((1, 8), lambda i: (0, 0)),
          pl.BlockSpec((HID, HID), lambda i: (0, 0)),
          pl.BlockSpec((HID, HID), lambda i: (0, 0)),
          pl.BlockSpec((1, HID), lambda i: (0, 0)),
      ],
      out_specs=[
          pl.BlockSpec((BN, 8), lambda i: (i, 0)),
          pl.BlockSpec((BN, HID), lambda i: (i, 0)),
          pl.BlockSpec((BN, HID), lambda i: (i, 0)),
      ],
      out_shape=[
          jax.ShapeDtypeStruct((NP, 8), jnp.float32),
          jax.ShapeDtypeStruct((NP, HID), jnp.float32),
          jax.ShapeDtypeStruct((NP, HID), jnp.float32),
      ],
  )(q0, q1, h1, p0, p1, wl2t, wr2t, b2row, wv1t, bv1row, wv2t8, bv2row8,
    we1at, we1bt, be1row)


def kernel(x, edge_index, edge_attr, Wl1, Wr1, b1, Wl2, Wr2, b2, Wg, att_src,
           att_dst, bg, Wv1, bv1, Wv2, bv2, We1, be1, We2, be2):
  del edge_attr, Wg, att_src, att_dst, bg
  src2d = edge_index[0].reshape(E // CH, CH)
  dst2d = edge_index[1].reshape(E // CH, CH)

  ones = jnp.ones((N, 1), jnp.float32)
  zeros8 = jnp.zeros((N, 8), jnp.float32)
  x_aug = jnp.concatenate([x, ones, zeros8], axis=1)
  x_aug = jnp.pad(x_aug, ((0, NP - N), (0, 0)))

  z16 = jnp.zeros((NP, 16), jnp.float32)
  z128 = jnp.zeros((NP, HID), jnp.float32)

  pad9 = jnp.zeros((9, HID), jnp.float32)
  wl1t = jnp.concatenate([Wl1.T, pad9], axis=0)
  wr1t = jnp.concatenate([Wr1.T, pad9], axis=0)
  b1row = b1.reshape(1, HID)
  wl2t = Wl2.T
  wr2t = Wr2.T
  b2row = b2.reshape(1, HID)
  wv1t = Wv1.T
  bv1row = bv1.reshape(1, HID)
  wv2t8 = jnp.concatenate([Wv2.T, jnp.zeros((HID, 5), jnp.float32)], axis=1)
  bv2row8 = jnp.concatenate([bv2, jnp.zeros((5,), jnp.float32)]).reshape(1, 8)
  we1at = We1[:, :HID].T
  we1bt = We1[:, HID:].T
  be1row = be1.reshape(1, HID)
  w_ext = jnp.concatenate([We2.reshape(HID), jnp.broadcast_to(be2, (16,))])

  p = _sc_segsum16(x_aug, src2d, dst2d, z16)
  p0, p1 = p[0], p[1]
  h1 = _tc_sage1(p0, p1, x_aug, wl1t, wr1t, b1row)
  q = _sc_segsum128(h1, src2d, dst2d, z128)
  pos8, a_tab, b_tab = _tc_sage2(q[0], q[1], h1, p0, p1, wl2t, wr2t, b2row,
                                 wv1t, bv1row, wv2t8, bv2row8, we1at, we1bt,
                                 be1row)
  conn = _sc_edge_mlp(a_tab, b_tab, src2d, dst2d, w_ext).reshape(E, 1)

  return pos8[:N, :3], conn


# speedup vs baseline: 3.8242x; 1.0458x over previous
import functools

import jax
import jax.numpy as jnp
from jax import lax
from jax.experimental import pallas as pl
from jax.experimental.pallas import tpu as pltpu
from jax.experimental.pallas import tpu_sc as plsc

N = 10000
NP = 10240
E = 320000
HID = 128

NC = 2
NS = 16
NW = NC * NS
EPT = E // NW
CH = 80
NCHUNK = EPT // CH
ROWS_PER_TILE = NP // NS

_MESH = plsc.VectorSubcoreMesh(
    core_axis_name="c", subcore_axis_name="s", num_cores=NC, num_subcores=NS)


def _make_sc_segsum(D, NSLOT):
  @functools.partial(
      pl.kernel,
      out_type=jax.ShapeDtypeStruct((NC, NP, D), jnp.float32),
      mesh=_MESH,
      scratch_types=[
          pltpu.VMEM((NCHUNK, CH), jnp.int32),
          pltpu.VMEM((NCHUNK, CH), jnp.int32),
          [pltpu.VMEM((CH, D), jnp.float32)] * NSLOT,
          pltpu.VMEM_SHARED((NP, D), jnp.float32),
          [pltpu.SemaphoreType.DMA] * NSLOT,
      ],
      compiler_params=pltpu.CompilerParams(use_tc_tiling_on_sc=False),
  )
  def segsum(table_hbm, src2d_hbm, dst2d_hbm, out_hbm,
             src_v, dst_v, rows, acc, sems):
    cid = lax.axis_index("c")
    sid = lax.axis_index("s")
    wid = sid * NC + cid
    r0 = sid * ROWS_PER_TILE
    c0 = wid * NCHUNK

    pltpu.sync_copy(src2d_hbm.at[pl.ds(c0, NCHUNK)], src_v)
    pltpu.sync_copy(dst2d_hbm.at[pl.ds(c0, NCHUNK)], dst_v)
    zv = jnp.zeros((16,), jnp.float32)

    def zbody(i, carry):
      rows[0][i // (D // 16), pl.ds((i % (D // 16)) * 16, 16)] = zv
      return carry

    lax.fori_loop(0, CH * D // 16, zbody, 0)
    for t in range(ROWS_PER_TILE // CH):
      pltpu.sync_copy(rows[0], acc.at[pl.ds(r0 + t * CH, CH)])

    def gather(c, s):
      return pltpu.async_copy(table_hbm.at[src_v.at[c]], rows[s], sems[s])

    for s in range(NSLOT):
      gather(s, s)
    plsc.subcore_barrier()

    def body(i, carry):
      for s in range(NSLOT):
        c = i * NSLOT + s
        pltpu.make_async_copy(table_hbm.at[src_v.at[c]], rows[s],
                              sems[s]).wait()
        pltpu.sync_copy(rows[s], acc.at[dst_v.at[c]], add=True)

        @pl.when(c + NSLOT < NCHUNK)
        def _():
          gather(c + NSLOT, s)

      return carry

    lax.fori_loop(0, NCHUNK // NSLOT, body, 0)
    for c in range(NCHUNK - NCHUNK % NSLOT, NCHUNK):
      s = c % NSLOT
      pltpu.make_async_copy(table_hbm.at[src_v.at[c]], rows[s], sems[s]).wait()
      pltpu.sync_copy(rows[s], acc.at[dst_v.at[c]], add=True)

    plsc.subcore_barrier()
    pltpu.sync_copy(acc.at[pl.ds(r0, ROWS_PER_TILE)],
                    out_hbm.at[cid, pl.ds(r0, ROWS_PER_TILE)])

  return segsum


_sc_segsum16 = _make_sc_segsum(16, 4)
_sc_segsum128 = _make_sc_segsum(HID, 2)


NSLOT_E = 5


@functools.partial(
    pl.kernel,
    out_type=jax.ShapeDtypeStruct((E,), jnp.float32),
    mesh=_MESH,
    scratch_types=[
        pltpu.VMEM((NCHUNK, CH), jnp.int32),
        pltpu.VMEM((NCHUNK, CH), jnp.int32),
        pltpu.VMEM((HID + 16,), jnp.float32),
        [pltpu.VMEM((CH, HID), jnp.float32)] * NSLOT_E,
        [pltpu.VMEM((CH, HID), jnp.float32)] * NSLOT_E,
        [pltpu.VMEM((CH,), jnp.float32)] * NSLOT_E,
        [pltpu.SemaphoreType.DMA] * NSLOT_E,
        [pltpu.SemaphoreType.DMA] * NSLOT_E,
    ],
    compiler_params=pltpu.CompilerParams(
        use_tc_tiling_on_sc=False, needs_layout_passes=False),
)
def _sc_edge_mlp(a_hbm, b_hbm, src2d_hbm, dst2d_hbm, w_hbm, out_hbm,
                 src_v, dst_v, w_v, a_bufs, b_bufs, o_bufs, sems_a, sems_b):
  cid = lax.axis_index("c")
  sid = lax.axis_index("s")
  wid = sid * NC + cid
  base = wid * EPT
  c0 = wid * NCHUNK

  pltpu.sync_copy(src2d_hbm.at[pl.ds(c0, NCHUNK)], src_v)
  pltpu.sync_copy(dst2d_hbm.at[pl.ds(c0, NCHUNK)], dst_v)
  pltpu.sync_copy(w_hbm, w_v)

  def gathers(c, s):
    pltpu.async_copy(a_hbm.at[src_v.at[c]], a_bufs[s], sems_a[s])
    pltpu.async_copy(b_hbm.at[dst_v.at[c]], b_bufs[s], sems_b[s])

  for s in range(NSLOT_E):
    gathers(s, s)

  w_regs = [w_v[pl.ds(j * 16, 16)] for j in range(HID // 16)]
  be2_vec = w_v[pl.ds(HID, 16)]

  def process(c, s):
    pltpu.make_async_copy(a_hbm.at[src_v.at[c]], a_bufs[s], sems_a[s]).wait()
    pltpu.make_async_copy(b_hbm.at[dst_v.at[c]], b_bufs[s], sems_b[s]).wait()

    lane = lax.iota(jnp.int32, 16)

    def group_body(g, carry2):
      gvec = be2_vec
      for u in range(16):
        r = g * 16 + u
        acc = None
        for j in range(HID // 16):
          sl = pl.ds(j * 16, 16)
          t = jnp.maximum(a_bufs[s][r, sl] + b_bufs[s][r, sl], 0.0) * w_regs[j]
          acc = t if acc is None else acc + t
        gvec = jnp.where(lane == u, gvec + jnp.sum(acc), gvec)
      o_bufs[s][pl.ds(g * 16, 16)] = 1.0 / (1.0 + jnp.exp(-gvec))
      return carry2

    lax.fori_loop(0, CH // 16, group_body, 0)
    pltpu.sync_copy(o_bufs[s], out_hbm.at[pl.ds(base + c * CH, CH)])

  def body(i, carry):
    for s in range(NSLOT_E):
      c = i * NSLOT_E + s
      process(c, s)

      @pl.when(c + NSLOT_E < NCHUNK)
      def _():
        gathers(c + NSLOT_E, s)

    return carry

  lax.fori_loop(0, NCHUNK // NSLOT_E, body, 0)
  for c in range(NCHUNK - NCHUNK % NSLOT_E, NCHUNK):
    process(c, c % NSLOT_E)


def _tc_sage1(p, x_aug, wl1t, wr1t, b1row):
  BN = 640
  grid = (NP // BN,)

  def body(p0_ref, p1_ref, x_ref, wl_ref, wr_ref, b_ref, o_ref):
    p_ = p0_ref[0] + p1_ref[0]
    deg = jnp.maximum(p_[:, 7:8], 1.0)
    agg = p_ / deg
    acc = jnp.dot(agg, wl_ref[...], preferred_element_type=jnp.float32)
    acc += jnp.dot(x_ref[...], wr_ref[...], preferred_element_type=jnp.float32)
    o_ref[...] = jnp.maximum(acc + b_ref[...], 0.0)

  return pl.pallas_call(
      body,
      grid=grid,
      in_specs=[
          pl.BlockSpec((1, BN, 16), lambda i: (0, i, 0)),
          pl.BlockSpec((1, BN, 16), lambda i: (1, i, 0)),
          pl.BlockSpec((BN, 16), lambda i: (i, 0)),
          pl.BlockSpec((16, HID), lambda i: (0, 0)),
          pl.BlockSpec((16, HID), lambda i: (0, 0)),
          pl.BlockSpec((1, HID), lambda i: (0, 0)),
      ],
      out_specs=pl.BlockSpec((BN, HID), lambda i: (i, 0)),
      out_shape=jax.ShapeDtypeStruct((NP, HID), jnp.float32),
  )(p, p, x_aug, wl1t, wr1t, b1row)


def _tc_sage2(q, h1, p, wl2t, wr2t, b2row, wv1t, bv1row, wv2t8,
              bv2row8, we1at, we1bt, be1row):
  BN = 640
  grid = (NP // BN,)

  def body(q0_ref, q1_ref, h1_ref, p0_ref, p1_ref, wl_ref, wr_ref, b2_ref,
           wv1_ref, bv1_ref, wv2_ref, bv2_ref, wea_ref, web_ref, be1_ref,
           pos_ref, a_ref, bm_ref):
    p_ = p0_ref[0] + p1_ref[0]
    deg = jnp.maximum(p_[:, 7:8], 1.0)
    q_ = (q0_ref[0] + q1_ref[0]) / deg
    h1 = h1_ref[...]
    h = jnp.dot(q_, wl_ref[...], preferred_element_type=jnp.float32)
    h += jnp.dot(h1, wr_ref[...], preferred_element_type=jnp.float32)
    h += b2_ref[...]
    t = jnp.maximum(
        jnp.dot(h, wv1_ref[...], preferred_element_type=jnp.float32)
        + bv1_ref[...], 0.0)
    pos_ref[...] = jnp.dot(
        t, wv2_ref[...], preferred_element_type=jnp.float32) + bv2_ref[...]
    a_ref[...] = jnp.dot(
        h, wea_ref[...], preferred_element_type=jnp.float32) + be1_ref[...]
    bm_ref[...] = jnp.dot(
        h, web_ref[...], preferred_element_type=jnp.float32)

  return pl.pallas_call(
      body,
      grid=grid,
      in_specs=[
          pl.BlockSpec((1, BN, HID), lambda i: (0, i, 0)),
          pl.BlockSpec((1, BN, HID), lambda i: (1, i, 0)),
          pl.BlockSpec((BN, HID), lambda i: (i, 0)),
          pl.BlockSpec((1, BN, 16), lambda i: (0, i, 0)),
          pl.BlockSpec((1, BN, 16), lambda i: (1, i, 0)),
          pl.BlockSpec((HID, HID), lambda i: (0, 0)),
          pl.BlockSpec((HID, HID), lambda i: (0, 0)),
          pl.BlockSpec((1, HID), lambda i: (0, 0)),
          pl.BlockSpec((HID, HID), lambda i: (0, 0)),
          pl.BlockSpec((1, HID), lambda i: (0, 0)),
          pl.BlockSpec((HID, 8), lambda i: (0, 0)),
          pl.BlockSpec((1, 8), lambda i: (0, 0)),
          pl.BlockSpec((HID, HID), lambda i: (0, 0)),
          pl.BlockSpec((HID, HID), lambda i: (0, 0)),
          pl.BlockSpec((1, HID), lambda i: (0, 0)),
      ],
      out_specs=[
          pl.BlockSpec((BN, 8), lambda i: (i, 0)),
          pl.BlockSpec((BN, HID), lambda i: (i, 0)),
          pl.BlockSpec((BN, HID), lambda i: (i, 0)),
      ],
      out_shape=[
          jax.ShapeDtypeStruct((NP, 8), jnp.float32),
          jax.ShapeDtypeStruct((NP, HID), jnp.float32),
          jax.ShapeDtypeStruct((NP, HID), jnp.float32),
      ],
  )(q, q, h1, p, p, wl2t, wr2t, b2row, wv1t, bv1row, wv2t8, bv2row8,
    we1at, we1bt, be1row)


def kernel(x, edge_index, edge_attr, Wl1, Wr1, b1, Wl2, Wr2, b2, Wg, att_src,
           att_dst, bg, Wv1, bv1, Wv2, bv2, We1, be1, We2, be2):
  del edge_attr, Wg, att_src, att_dst, bg
  src2d = edge_index[0].reshape(E // CH, CH)
  dst2d = edge_index[1].reshape(E // CH, CH)

  ones = jnp.ones((N, 1), jnp.float32)
  zeros8 = jnp.zeros((N, 8), jnp.float32)
  x_aug = jnp.concatenate([x, ones, zeros8], axis=1)
  x_aug = jnp.pad(x_aug, ((0, NP - N), (0, 0)))

  pad9 = jnp.zeros((9, HID), jnp.float32)
  wl1t = jnp.concatenate([Wl1.T, pad9], axis=0)
  wr1t = jnp.concatenate([Wr1.T, pad9], axis=0)
  b1row = b1.reshape(1, HID)
  wl2t = Wl2.T
  wr2t = Wr2.T
  b2row = b2.reshape(1, HID)
  wv1t = Wv1.T
  bv1row = bv1.reshape(1, HID)
  wv2t8 = jnp.concatenate([Wv2.T, jnp.zeros((HID, 5), jnp.float32)], axis=1)
  bv2row8 = jnp.concatenate([bv2, jnp.zeros((5,), jnp.float32)]).reshape(1, 8)
  we1at = We1[:, :HID].T
  we1bt = We1[:, HID:].T
  be1row = be1.reshape(1, HID)
  w_ext = jnp.concatenate([We2.reshape(HID), jnp.broadcast_to(be2, (16,))])

  p = _sc_segsum16(x_aug, src2d, dst2d)
  h1 = _tc_sage1(p, x_aug, wl1t, wr1t, b1row)
  q = _sc_segsum128(h1, src2d, dst2d)
  pos8, a_tab, b_tab = _tc_sage2(q, h1, p, wl2t, wr2t, b2row,
                                 wv1t, bv1row, wv2t8, bv2row8, we1at, we1bt,
                                 be1row)
  conn = _sc_edge_mlp(a_tab, b_tab, src2d, dst2d, w_ext).reshape(E, 1)

  return pos8[:N, :3], conn


# speedup vs baseline: 4.3721x; 1.1433x over previous
import functools

import jax
import jax.numpy as jnp
from jax import lax
from jax.experimental import pallas as pl
from jax.experimental.pallas import tpu as pltpu
from jax.experimental.pallas import tpu_sc as plsc

N = 10000
NP = 10240
E = 320000
HID = 128

NC = 2
NS = 16
NW = NC * NS
EPT = E // NW
CH = 80
NCHUNK = EPT // CH
ROWS_PER_TILE = NP // NS

_MESH = plsc.VectorSubcoreMesh(
    core_axis_name="c", subcore_axis_name="s", num_cores=NC, num_subcores=NS)


def _make_sc_segsum(D, NSLOT):
  @functools.partial(
      pl.kernel,
      out_type=jax.ShapeDtypeStruct((NC, NP, D), jnp.float32),
      mesh=_MESH,
      scratch_types=[
          pltpu.VMEM((NCHUNK, CH), jnp.int32),
          pltpu.VMEM((NCHUNK, CH), jnp.int32),
          [pltpu.VMEM((CH, D), jnp.float32)] * NSLOT,
          pltpu.VMEM_SHARED((NP, D), jnp.float32),
          [pltpu.SemaphoreType.DMA] * NSLOT,
      ],
      compiler_params=pltpu.CompilerParams(use_tc_tiling_on_sc=False),
  )
  def segsum(table_hbm, src2d_hbm, dst2d_hbm, out_hbm,
             src_v, dst_v, rows, acc, sems):
    cid = lax.axis_index("c")
    sid = lax.axis_index("s")
    wid = sid * NC + cid
    r0 = sid * ROWS_PER_TILE
    c0 = wid * NCHUNK

    pltpu.sync_copy(src2d_hbm.at[pl.ds(c0, NCHUNK)], src_v)
    pltpu.sync_copy(dst2d_hbm.at[pl.ds(c0, NCHUNK)], dst_v)
    zv = jnp.zeros((16,), jnp.float32)

    def zbody(i, carry):
      rows[0][i // (D // 16), pl.ds((i % (D // 16)) * 16, 16)] = zv
      return carry

    lax.fori_loop(0, CH * D // 16, zbody, 0)
    for t in range(ROWS_PER_TILE // CH):
      pltpu.sync_copy(rows[0], acc.at[pl.ds(r0 + t * CH, CH)])

    def gather(c, s):
      return pltpu.async_copy(table_hbm.at[src_v.at[c]], rows[s], sems[s])

    for s in range(NSLOT):
      gather(s, s)
    plsc.subcore_barrier()

    def body(i, carry):
      for s in range(NSLOT):
        c = i * NSLOT + s
        pltpu.make_async_copy(table_hbm.at[src_v.at[c]], rows[s],
                              sems[s]).wait()
        pltpu.sync_copy(rows[s], acc.at[dst_v.at[c]], add=True)

        @pl.when(c + NSLOT < NCHUNK)
        def _():
          gather(c + NSLOT, s)

      return carry

    lax.fori_loop(0, NCHUNK // NSLOT, body, 0)
    for c in range(NCHUNK - NCHUNK % NSLOT, NCHUNK):
      s = c % NSLOT
      pltpu.make_async_copy(table_hbm.at[src_v.at[c]], rows[s], sems[s]).wait()
      pltpu.sync_copy(rows[s], acc.at[dst_v.at[c]], add=True)

    plsc.subcore_barrier()
    pltpu.sync_copy(acc.at[pl.ds(r0, ROWS_PER_TILE)],
                    out_hbm.at[cid, pl.ds(r0, ROWS_PER_TILE)])

  return segsum


_sc_segsum16 = _make_sc_segsum(16, 4)
_sc_segsum128 = _make_sc_segsum(HID, 2)


NSLOT_E = 5


@functools.partial(
    pl.kernel,
    out_type=jax.ShapeDtypeStruct((E,), jnp.float32),
    mesh=_MESH,
    scratch_types=[
        pltpu.VMEM((NCHUNK, CH), jnp.int32),
        pltpu.VMEM((NCHUNK, CH), jnp.int32),
        pltpu.VMEM((HID + 16,), jnp.float32),
        pltpu.VMEM((HID,), jnp.bfloat16),
        [pltpu.VMEM((CH, HID), jnp.bfloat16)] * NSLOT_E,
        [pltpu.VMEM((CH, HID), jnp.bfloat16)] * NSLOT_E,
        [pltpu.VMEM((CH,), jnp.float32)] * NSLOT_E,
        [pltpu.SemaphoreType.DMA] * NSLOT_E,
        [pltpu.SemaphoreType.DMA] * NSLOT_E,
    ],
    compiler_params=pltpu.CompilerParams(
        use_tc_tiling_on_sc=False, needs_layout_passes=False),
)
def _sc_edge_mlp(a_hbm, b_hbm, src2d_hbm, dst2d_hbm, w_hbm, wbf_hbm, out_hbm,
                 src_v, dst_v, w_v, wbf_v, a_bufs, b_bufs, o_bufs,
                 sems_a, sems_b):
  cid = lax.axis_index("c")
  sid = lax.axis_index("s")
  wid = sid * NC + cid
  base = wid * EPT
  c0 = wid * NCHUNK

  pltpu.sync_copy(src2d_hbm.at[pl.ds(c0, NCHUNK)], src_v)
  pltpu.sync_copy(dst2d_hbm.at[pl.ds(c0, NCHUNK)], dst_v)
  pltpu.sync_copy(w_hbm, w_v)
  pltpu.sync_copy(wbf_hbm, wbf_v)

  def gathers(c, s):
    pltpu.async_copy(a_hbm.at[src_v.at[c]], a_bufs[s], sems_a[s])
    pltpu.async_copy(b_hbm.at[dst_v.at[c]], b_bufs[s], sems_b[s])

  for s in range(NSLOT_E):
    gathers(s, s)

  w_regs = [wbf_v[pl.ds(j * 32, 32)] for j in range(HID // 32)]
  be2_vec = w_v[pl.ds(HID, 16)]

  def process(c, s):
    pltpu.make_async_copy(a_hbm.at[src_v.at[c]], a_bufs[s], sems_a[s]).wait()
    pltpu.make_async_copy(b_hbm.at[dst_v.at[c]], b_bufs[s], sems_b[s]).wait()

    lane = lax.iota(jnp.int32, 16)

    def group_body(g, carry2):
      gvec = be2_vec
      zero_bf = jnp.zeros((32,), jnp.bfloat16)
      for u in range(16):
        r = g * 16 + u
        acc = None
        for j in range(HID // 32):
          sl = pl.ds(j * 32, 32)
          t = jnp.maximum(a_bufs[s][r, sl] + b_bufs[s][r, sl],
                          zero_bf) * w_regs[j]
          t0, t1 = plsc.unpack(t, format=plsc.PackFormat.INTERLEAVED)
          acc = t0 + t1 if acc is None else acc + t0 + t1
        gvec = jnp.where(lane == u, gvec + jnp.sum(acc), gvec)
      o_bufs[s][pl.ds(g * 16, 16)] = 1.0 / (1.0 + jnp.exp(-gvec))
      return carry2

    lax.fori_loop(0, CH // 16, group_body, 0)
    pltpu.sync_copy(o_bufs[s], out_hbm.at[pl.ds(base + c * CH, CH)])

  def body(i, carry):
    for s in range(NSLOT_E):
      c = i * NSLOT_E + s
      process(c, s)

      @pl.when(c + NSLOT_E < NCHUNK)
      def _():
        gathers(c + NSLOT_E, s)

    return carry

  lax.fori_loop(0, NCHUNK // NSLOT_E, body, 0)
  for c in range(NCHUNK - NCHUNK % NSLOT_E, NCHUNK):
    process(c, c % NSLOT_E)


def _tc_sage1(p, x_aug, wl1t, wr1t, b1row):
  BN = 640
  grid = (NP // BN,)

  def body(p0_ref, p1_ref, x_ref, wl_ref, wr_ref, b_ref, o_ref):
    p_ = p0_ref[0] + p1_ref[0]
    deg = jnp.maximum(p_[:, 7:8], 1.0)
    agg = p_ / deg
    acc = jnp.dot(agg, wl_ref[...], preferred_element_type=jnp.float32)
    acc += jnp.dot(x_ref[...], wr_ref[...], preferred_element_type=jnp.float32)
    o_ref[...] = jnp.maximum(acc + b_ref[...], 0.0)

  return pl.pallas_call(
      body,
      grid=grid,
      in_specs=[
          pl.BlockSpec((1, BN, 16), lambda i: (0, i, 0)),
          pl.BlockSpec((1, BN, 16), lambda i: (1, i, 0)),
          pl.BlockSpec((BN, 16), lambda i: (i, 0)),
          pl.BlockSpec((16, HID), lambda i: (0, 0)),
          pl.BlockSpec((16, HID), lambda i: (0, 0)),
          pl.BlockSpec((1, HID), lambda i: (0, 0)),
      ],
      out_specs=pl.BlockSpec((BN, HID), lambda i: (i, 0)),
      out_shape=jax.ShapeDtypeStruct((NP, HID), jnp.float32),
  )(p, p, x_aug, wl1t, wr1t, b1row)


def _tc_sage2(q, h1, p, wl2t, wr2t, b2row, wv1t, bv1row, wv2t8,
              bv2row8, we1at, we1bt, be1row):
  BN = 640
  grid = (NP // BN,)

  def body(q0_ref, q1_ref, h1_ref, p0_ref, p1_ref, wl_ref, wr_ref, b2_ref,
           wv1_ref, bv1_ref, wv2_ref, bv2_ref, wea_ref, web_ref, be1_ref,
           pos_ref, a_ref, bm_ref):
    p_ = p0_ref[0] + p1_ref[0]
    deg = jnp.maximum(p_[:, 7:8], 1.0)
    q_ = (q0_ref[0] + q1_ref[0]) / deg
    h1 = h1_ref[...]
    h = jnp.dot(q_, wl_ref[...], preferred_element_type=jnp.float32)
    h += jnp.dot(h1, wr_ref[...], preferred_element_type=jnp.float32)
    h += b2_ref[...]
    t = jnp.maximum(
        jnp.dot(h, wv1_ref[...], preferred_element_type=jnp.float32)
        + bv1_ref[...], 0.0)
    pos_ref[...] = jnp.dot(
        t, wv2_ref[...], preferred_element_type=jnp.float32) + bv2_ref[...]
    a_ref[...] = (jnp.dot(
        h, wea_ref[...], preferred_element_type=jnp.float32)
        + be1_ref[...]).astype(jnp.bfloat16)
    bm_ref[...] = jnp.dot(
        h, web_ref[...], preferred_element_type=jnp.float32).astype(
            jnp.bfloat16)

  return pl.pallas_call(
      body,
      grid=grid,
      in_specs=[
          pl.BlockSpec((1, BN, HID), lambda i: (0, i, 0)),
          pl.BlockSpec((1, BN, HID), lambda i: (1, i, 0)),
          pl.BlockSpec((BN, HID), lambda i: (i, 0)),
          pl.BlockSpec((1, BN, 16), lambda i: (0, i, 0)),
          pl.BlockSpec((1, BN, 16), lambda i: (1, i, 0)),
          pl.BlockSpec((HID, HID), lambda i: (0, 0)),
          pl.BlockSpec((HID, HID), lambda i: (0, 0)),
          pl.BlockSpec((1, HID), lambda i: (0, 0)),
          pl.BlockSpec((HID, HID), lambda i: (0, 0)),
          pl.BlockSpec((1, HID), lambda i: (0, 0)),
          pl.BlockSpec((HID, 8), lambda i: (0, 0)),
          pl.BlockSpec((1, 8), lambda i: (0, 0)),
          pl.BlockSpec((HID, HID), lambda i: (0, 0)),
          pl.BlockSpec((HID, HID), lambda i: (0, 0)),
          pl.BlockSpec((1, HID), lambda i: (0, 0)),
      ],
      out_specs=[
          pl.BlockSpec((BN, 8), lambda i: (i, 0)),
          pl.BlockSpec((BN, HID), lambda i: (i, 0)),
          pl.BlockSpec((BN, HID), lambda i: (i, 0)),
      ],
      out_shape=[
          jax.ShapeDtypeStruct((NP, 8), jnp.float32),
          jax.ShapeDtypeStruct((NP, HID), jnp.bfloat16),
          jax.ShapeDtypeStruct((NP, HID), jnp.bfloat16),
      ],
  )(q, q, h1, p, p, wl2t, wr2t, b2row, wv1t, bv1row, wv2t8, bv2row8,
    we1at, we1bt, be1row)


def kernel(x, edge_index, edge_attr, Wl1, Wr1, b1, Wl2, Wr2, b2, Wg, att_src,
           att_dst, bg, Wv1, bv1, Wv2, bv2, We1, be1, We2, be2):
  del edge_attr, Wg, att_src, att_dst, bg
  src2d = edge_index[0].reshape(E // CH, CH)
  dst2d = edge_index[1].reshape(E // CH, CH)

  ones = jnp.ones((N, 1), jnp.float32)
  zeros8 = jnp.zeros((N, 8), jnp.float32)
  x_aug = jnp.concatenate([x, ones, zeros8], axis=1)
  x_aug = jnp.pad(x_aug, ((0, NP - N), (0, 0)))

  pad9 = jnp.zeros((9, HID), jnp.float32)
  wl1t = jnp.concatenate([Wl1.T, pad9], axis=0)
  wr1t = jnp.concatenate([Wr1.T, pad9], axis=0)
  b1row = b1.reshape(1, HID)
  wl2t = Wl2.T
  wr2t = Wr2.T
  b2row = b2.reshape(1, HID)
  wv1t = Wv1.T
  bv1row = bv1.reshape(1, HID)
  wv2t8 = jnp.concatenate([Wv2.T, jnp.zeros((HID, 5), jnp.float32)], axis=1)
  bv2row8 = jnp.concatenate([bv2, jnp.zeros((5,), jnp.float32)]).reshape(1, 8)
  we1at = We1[:, :HID].T
  we1bt = We1[:, HID:].T
  be1row = be1.reshape(1, HID)
  w_ext = jnp.concatenate([We2.reshape(HID), jnp.broadcast_to(be2, (16,))])
  w_bf = We2.reshape(HID).astype(jnp.bfloat16)

  p = _sc_segsum16(x_aug, src2d, dst2d)
  h1 = _tc_sage1(p, x_aug, wl1t, wr1t, b1row)
  q = _sc_segsum128(h1, src2d, dst2d)
  pos8, a_tab, b_tab = _tc_sage2(q, h1, p, wl2t, wr2t, b2row,
                                 wv1t, bv1row, wv2t8, bv2row8, we1at, we1bt,
                                 be1row)
  conn = _sc_edge_mlp(a_tab, b_tab, src2d, dst2d, w_ext, w_bf).reshape(E, 1)

  return pos8[:N, :3], conn


# speedup vs baseline: 4.5734x; 1.0461x over previous
import functools

import jax
import jax.numpy as jnp
from jax import lax
from jax.experimental import pallas as pl
from jax.experimental.pallas import tpu as pltpu
from jax.experimental.pallas import tpu_sc as plsc

N = 10000
NP = 10240
E = 320000
HID = 128

NC = 2
NS = 16
NW = NC * NS
EPT = E // NW
CH = 80
NCHUNK = EPT // CH
ROWS_PER_TILE = NP // NS

_MESH = plsc.VectorSubcoreMesh(
    core_axis_name="c", subcore_axis_name="s", num_cores=NC, num_subcores=NS)


def _make_sc_segsum(D, NSLOT):
  @functools.partial(
      pl.kernel,
      out_type=jax.ShapeDtypeStruct((NC, NP, D), jnp.float32),
      mesh=_MESH,
      scratch_types=[
          pltpu.VMEM((NCHUNK, CH), jnp.int32),
          [pltpu.VMEM((CH,), jnp.int32)] * NSLOT,
          [pltpu.VMEM((CH, D), jnp.float32)] * NSLOT,
          pltpu.VMEM_SHARED((NP, D), jnp.float32),
          [pltpu.SemaphoreType.DMA] * NSLOT,
          [pltpu.SemaphoreType.DMA] * NSLOT,
      ],
      compiler_params=pltpu.CompilerParams(use_tc_tiling_on_sc=False),
  )
  def segsum(table_hbm, src2d_hbm, dst2d_hbm, out_hbm,
             src_v, dst_rings, rows, acc, sems, dsems):
    cid = lax.axis_index("c")
    sid = lax.axis_index("s")
    wid = sid * NC + cid
    r0 = sid * ROWS_PER_TILE
    c0 = wid * NCHUNK

    pltpu.sync_copy(src2d_hbm.at[pl.ds(c0, NCHUNK)], src_v)
    zv = jnp.zeros((16,), jnp.float32)

    def zbody(i, carry):
      rows[0][i // (D // 16), pl.ds((i % (D // 16)) * 16, 16)] = zv
      return carry

    lax.fori_loop(0, CH * D // 16, zbody, 0)
    for t in range(ROWS_PER_TILE // CH):
      pltpu.sync_copy(rows[0], acc.at[pl.ds(r0 + t * CH, CH)])

    def gather(c, s):
      pltpu.async_copy(dst2d_hbm.at[c0 + c], dst_rings[s], dsems[s])
      pltpu.async_copy(table_hbm.at[src_v.at[c]], rows[s], sems[s])

    for s in range(NSLOT):
      gather(s, s)
    plsc.subcore_barrier()

    def process(c, s):
      pltpu.make_async_copy(dst2d_hbm.at[c0 + c], dst_rings[s],
                            dsems[s]).wait()
      pltpu.make_async_copy(table_hbm.at[src_v.at[c]], rows[s],
                            sems[s]).wait()
      pltpu.sync_copy(rows[s], acc.at[dst_rings[s]], add=True)

    def body(i, carry):
      for s in range(NSLOT):
        c = i * NSLOT + s
        process(c, s)

        @pl.when(c + NSLOT < NCHUNK)
        def _():
          gather(c + NSLOT, s)

      return carry

    lax.fori_loop(0, NCHUNK // NSLOT, body, 0)
    for c in range(NCHUNK - NCHUNK % NSLOT, NCHUNK):
      process(c, c % NSLOT)

    plsc.subcore_barrier()
    pltpu.sync_copy(acc.at[pl.ds(r0, ROWS_PER_TILE)],
                    out_hbm.at[cid, pl.ds(r0, ROWS_PER_TILE)])

  return segsum


_sc_segsum16 = _make_sc_segsum(16, 4)
_sc_segsum128 = _make_sc_segsum(HID, 3)


NSLOT_E = 8


@functools.partial(
    pl.kernel,
    out_type=jax.ShapeDtypeStruct((E,), jnp.float32),
    mesh=_MESH,
    scratch_types=[
        pltpu.VMEM((NCHUNK, CH), jnp.int32),
        pltpu.VMEM((NCHUNK, CH), jnp.int32),
        pltpu.VMEM((HID + 16,), jnp.float32),
        pltpu.VMEM((HID,), jnp.bfloat16),
        [pltpu.VMEM((CH, HID), jnp.bfloat16)] * NSLOT_E,
        [pltpu.VMEM((CH, HID), jnp.bfloat16)] * NSLOT_E,
        [pltpu.VMEM((CH,), jnp.float32)] * NSLOT_E,
        [pltpu.SemaphoreType.DMA] * NSLOT_E,
        [pltpu.SemaphoreType.DMA] * NSLOT_E,
    ],
    compiler_params=pltpu.CompilerParams(
        use_tc_tiling_on_sc=False, needs_layout_passes=False),
)
def _sc_edge_mlp(a_hbm, b_hbm, src2d_hbm, dst2d_hbm, w_hbm, wbf_hbm, out_hbm,
                 src_v, dst_v, w_v, wbf_v, a_bufs, b_bufs, o_bufs,
                 sems_a, sems_b):
  cid = lax.axis_index("c")
  sid = lax.axis_index("s")
  wid = sid * NC + cid
  base = wid * EPT
  c0 = wid * NCHUNK

  pltpu.sync_copy(src2d_hbm.at[pl.ds(c0, NCHUNK)], src_v)
  pltpu.sync_copy(dst2d_hbm.at[pl.ds(c0, NCHUNK)], dst_v)
  pltpu.sync_copy(w_hbm, w_v)
  pltpu.sync_copy(wbf_hbm, wbf_v)

  def gathers(c, s):
    pltpu.async_copy(a_hbm.at[src_v.at[c]], a_bufs[s], sems_a[s])
    pltpu.async_copy(b_hbm.at[dst_v.at[c]], b_bufs[s], sems_b[s])

  for s in range(NSLOT_E):
    gathers(s, s)

  w_regs = [wbf_v[pl.ds(j * 32, 32)] for j in range(HID // 32)]
  be2_vec = w_v[pl.ds(HID, 16)]

  def process(c, s):
    pltpu.make_async_copy(a_hbm.at[src_v.at[c]], a_bufs[s], sems_a[s]).wait()
    pltpu.make_async_copy(b_hbm.at[dst_v.at[c]], b_bufs[s], sems_b[s]).wait()

    lane = lax.iota(jnp.int32, 16)

    def group_body(g, carry2):
      gvec = be2_vec
      zero_bf = jnp.zeros((32,), jnp.bfloat16)
      for u in range(16):
        r = g * 16 + u
        acc = None
        for j in range(HID // 32):
          sl = pl.ds(j * 32, 32)
          t = jnp.maximum(a_bufs[s][r, sl] + b_bufs[s][r, sl],
                          zero_bf) * w_regs[j]
          t0, t1 = plsc.unpack(t, format=plsc.PackFormat.INTERLEAVED)
          acc = t0 + t1 if acc is None else acc + t0 + t1
        gvec = jnp.where(lane == u, gvec + jnp.sum(acc), gvec)
      o_bufs[s][pl.ds(g * 16, 16)] = 1.0 / (1.0 + jnp.exp(-gvec))
      return carry2

    lax.fori_loop(0, CH // 16, group_body, 0)
    pltpu.sync_copy(o_bufs[s], out_hbm.at[pl.ds(base + c * CH, CH)])

  def body(i, carry):
    for s in range(NSLOT_E):
      c = i * NSLOT_E + s
      process(c, s)

      @pl.when(c + NSLOT_E < NCHUNK)
      def _():
        gathers(c + NSLOT_E, s)

    return carry

  lax.fori_loop(0, NCHUNK // NSLOT_E, body, 0)
  for c in range(NCHUNK - NCHUNK % NSLOT_E, NCHUNK):
    process(c, c % NSLOT_E)


def _tc_sage1(p, x_aug, wl1t, wr1t, b1row):
  BN = 640
  grid = (NP // BN,)

  def body(p0_ref, p1_ref, x_ref, wl_ref, wr_ref, b_ref, o_ref):
    p_ = p0_ref[0] + p1_ref[0]
    deg = jnp.maximum(p_[:, 7:8], 1.0)
    agg = p_ / deg
    acc = jnp.dot(agg, wl_ref[...], preferred_element_type=jnp.float32)
    acc += jnp.dot(x_ref[...], wr_ref[...], preferred_element_type=jnp.float32)
    o_ref[...] = jnp.maximum(acc + b_ref[...], 0.0)

  return pl.pallas_call(
      body,
      grid=grid,
      in_specs=[
          pl.BlockSpec((1, BN, 16), lambda i: (0, i, 0)),
          pl.BlockSpec((1, BN, 16), lambda i: (1, i, 0)),
          pl.BlockSpec((BN, 16), lambda i: (i, 0)),
          pl.BlockSpec((16, HID), lambda i: (0, 0)),
          pl.BlockSpec((16, HID), lambda i: (0, 0)),
          pl.BlockSpec((1, HID), lambda i: (0, 0)),
      ],
      out_specs=pl.BlockSpec((BN, HID), lambda i: (i, 0)),
      out_shape=jax.ShapeDtypeStruct((NP, HID), jnp.float32),
  )(p, p, x_aug, wl1t, wr1t, b1row)


def _tc_sage2(q, h1, p, wl2t, wr2t, b2row, wv1t, bv1row, wv2t8,
              bv2row8, we1at, we1bt, be1row):
  BN = 640
  grid = (NP // BN,)

  def body(q0_ref, q1_ref, h1_ref, p0_ref, p1_ref, wl_ref, wr_ref, b2_ref,
           wv1_ref, bv1_ref, wv2_ref, bv2_ref, wea_ref, web_ref, be1_ref,
           pos_ref, a_ref, bm_ref):
    p_ = p0_ref[0] + p1_ref[0]
    deg = jnp.maximum(p_[:, 7:8], 1.0)
    q_ = (q0_ref[0] + q1_ref[0]) / deg
    h1 = h1_ref[...]
    h = jnp.dot(q_, wl_ref[...], preferred_element_type=jnp.float32)
    h += jnp.dot(h1, wr_ref[...], preferred_element_type=jnp.float32)
    h += b2_ref[...]
    t = jnp.maximum(
        jnp.dot(h, wv1_ref[...], preferred_element_type=jnp.float32)
        + bv1_ref[...], 0.0)
    pos_ref[...] = jnp.dot(
        t, wv2_ref[...], preferred_element_type=jnp.float32) + bv2_ref[...]
    a_ref[...] = (jnp.dot(
        h, wea_ref[...], preferred_element_type=jnp.float32)
        + be1_ref[...]).astype(jnp.bfloat16)
    bm_ref[...] = jnp.dot(
        h, web_ref[...], preferred_element_type=jnp.float32).astype(
            jnp.bfloat16)

  return pl.pallas_call(
      body,
      grid=grid,
      in_specs=[
          pl.BlockSpec((1, BN, HID), lambda i: (0, i, 0)),
          pl.BlockSpec((1, BN, HID), lambda i: (1, i, 0)),
          pl.BlockSpec((BN, HID), lambda i: (i, 0)),
          pl.BlockSpec((1, BN, 16), lambda i: (0, i, 0)),
          pl.BlockSpec((1, BN, 16), lambda i: (1, i, 0)),
          pl.BlockSpec((HID, HID), lambda i: (0, 0)),
          pl.BlockSpec((HID, HID), lambda i: (0, 0)),
          pl.BlockSpec((1, HID), lambda i: (0, 0)),
          pl.BlockSpec((HID, HID), lambda i: (0, 0)),
          pl.BlockSpec((1, HID), lambda i: (0, 0)),
          pl.BlockSpec((HID, 8), lambda i: (0, 0)),
          pl.BlockSpec((1, 8), lambda i: (0, 0)),
          pl.BlockSpec((HID, HID), lambda i: (0, 0)),
          pl.BlockSpec((HID, HID), lambda i: (0, 0)),
          pl.BlockSpec((1, HID), lambda i: (0, 0)),
      ],
      out_specs=[
          pl.BlockSpec((BN, 8), lambda i: (i, 0)),
          pl.BlockSpec((BN, HID), lambda i: (i, 0)),
          pl.BlockSpec((BN, HID), lambda i: (i, 0)),
      ],
      out_shape=[
          jax.ShapeDtypeStruct((NP, 8), jnp.float32),
          jax.ShapeDtypeStruct((NP, HID), jnp.bfloat16),
          jax.ShapeDtypeStruct((NP, HID), jnp.bfloat16),
      ],
  )(q, q, h1, p, p, wl2t, wr2t, b2row, wv1t, bv1row, wv2t8, bv2row8,
    we1at, we1bt, be1row)


def kernel(x, edge_index, edge_attr, Wl1, Wr1, b1, Wl2, Wr2, b2, Wg, att_src,
           att_dst, bg, Wv1, bv1, Wv2, bv2, We1, be1, We2, be2):
  del edge_attr, Wg, att_src, att_dst, bg
  src2d = edge_index[0].reshape(E // CH, CH)
  dst2d = edge_index[1].reshape(E // CH, CH)

  ones = jnp.ones((N, 1), jnp.float32)
  zeros8 = jnp.zeros((N, 8), jnp.float32)
  x_aug = jnp.concatenate([x, ones, zeros8], axis=1)
  x_aug = jnp.pad(x_aug, ((0, NP - N), (0, 0)))

  pad9 = jnp.zeros((9, HID), jnp.float32)
  wl1t = jnp.concatenate([Wl1.T, pad9], axis=0)
  wr1t = jnp.concatenate([Wr1.T, pad9], axis=0)
  b1row = b1.reshape(1, HID)
  wl2t = Wl2.T
  wr2t = Wr2.T
  b2row = b2.reshape(1, HID)
  wv1t = Wv1.T
  bv1row = bv1.reshape(1, HID)
  wv2t8 = jnp.concatenate([Wv2.T, jnp.zeros((HID, 5), jnp.float32)], axis=1)
  bv2row8 = jnp.concatenate([bv2, jnp.zeros((5,), jnp.float32)]).reshape(1, 8)
  we1at = We1[:, :HID].T
  we1bt = We1[:, HID:].T
  be1row = be1.reshape(1, HID)
  w_ext = jnp.concatenate([We2.reshape(HID), jnp.broadcast_to(be2, (16,))])
  w_bf = We2.reshape(HID).astype(jnp.bfloat16)

  p = _sc_segsum16(x_aug, src2d, dst2d)
  h1 = _tc_sage1(p, x_aug, wl1t, wr1t, b1row)
  q = _sc_segsum128(h1, src2d, dst2d)
  pos8, a_tab, b_tab = _tc_sage2(q, h1, p, wl2t, wr2t, b2row,
                                 wv1t, bv1row, wv2t8, bv2row8, we1at, we1bt,
                                 be1row)
  conn = _sc_edge_mlp(a_tab, b_tab, src2d, dst2d, w_ext, w_bf).reshape(E, 1)

  return pos8[:N, :3], conn
